# baseline (speedup 1.0000x reference)
import jax
import jax.numpy as jnp
from jax import lax
from jax.experimental import pallas as pl
from jax.experimental.pallas import tpu as pltpu
from jax.experimental.pallas import tpu_sc as plsc

N_ROWS = 100000
DIM = 128
N_GRAPHS = 512

NC = 2
NS = 16
NW = NC * NS

CHUNK = 128
NCH = (N_ROWS + CHUNK - 1) // CHUNK
TAIL_CHUNK = NCH - 1
TAIL_ROWS = N_ROWS - TAIL_CHUNK * CHUNK
MAX_CH = 25
IDX_PER_W = MAX_CH * CHUNK
TAIL_IDX = N_ROWS - (NW - 1) * IDX_PER_W
NBUF = 6


def _sc_body(x_hbm, b_hbm, out_hbm, idx_raw, idx_v, xbuf, zbuf, acc, gsems, ssems):
    c = lax.axis_index("c")
    s = lax.axis_index("s")
    w = s * NC + c

    start = w * MAX_CH
    count = jnp.clip(NCH - start, 0, MAX_CH)
    n_full = count - (w == NW - 1).astype(jnp.int32)

    def gather_start(j, b):
        pltpu.async_copy(
            x_hbm.at[pl.ds((start + j) * CHUNK, CHUNK)], xbuf.at[b], gsems.at[b]
        )

    def gather_wait(j, b):
        pltpu.make_async_copy(
            x_hbm.at[pl.ds((start + j) * CHUNK, CHUNK)], xbuf.at[b], gsems.at[b]
        ).wait()

    def scatter_start(j, b):
        pltpu.async_copy(xbuf.at[b], acc.at[idx_v.at[j, 0]], ssems.at[b], add=True)

    def scatter_wait(j, b):
        pltpu.make_async_copy(
            xbuf.at[b], acc.at[idx_v.at[j, 0]], ssems.at[b]
        ).wait()

    gather_start(0, 0)
    gather_start(1, 1)
    gather_start(2, 2)
    gather_start(3, 3)

    @pl.when(w == NW - 1)
    def _():
        pltpu.sync_copy(
            b_hbm.at[pl.ds((NW - 1) * IDX_PER_W, TAIL_IDX)],
            idx_raw.at[pl.ds(0, TAIL_IDX)],
        )

    @pl.when(w < NW - 1)
    def _():
        pltpu.sync_copy(b_hbm.at[pl.ds(w * IDX_PER_W, IDX_PER_W)], idx_raw)

    ziv = jnp.zeros((16,), jnp.int32)

    @pl.when(w == NW - 1)
    def _():
        def zidx(t, carry):
            idx_raw[pl.ds(TAIL_IDX + t * 16, 16)] = ziv
            return carry

        lax.fori_loop(0, (IDX_PER_W - TAIL_IDX) // 16, zidx, 0)

    def fixup(j, carry):
        for m in range(CHUNK // 16):
            idx_v[j, 0, pl.ds(m * 16, 16)] = idx_raw[pl.ds(j * CHUNK + m * 16, 16)]
        return carry

    lax.fori_loop(0, MAX_CH, fixup, 0)

    zv = jnp.zeros((16,), jnp.float32)

    def zrow(r, carry):
        for k in range(DIM // 16):
            zbuf[r, pl.ds(k * 16, 16)] = zv
        return carry

    lax.fori_loop(0, 32, zrow, 0)
    pltpu.sync_copy(zbuf, acc.at[pl.ds(s * 32, 32)])
    plsc.subcore_barrier()

    def body(j, carry):
        jn = j + 4

        @pl.when(jn < n_full)
        def _():
            bn = jn % NBUF

            @pl.when(jn >= NBUF)
            def _():
                scatter_wait(jn - NBUF, bn)

            gather_start(jn, bn)

        b = j % NBUF
        gather_wait(j, b)
        scatter_start(j, b)
        return carry

    lax.fori_loop(0, n_full, body, 0)

    def drain(t, carry):
        j = n_full - NBUF + t

        @pl.when(j >= 0)
        def _():
            scatter_wait(j, j % NBUF)

        return carry

    lax.fori_loop(0, NBUF, drain, 0)

    @pl.when(w == NW - 1)
    def _():
        pltpu.sync_copy(
            x_hbm.at[pl.ds(TAIL_CHUNK * CHUNK, TAIL_ROWS)],
            xbuf.at[0, pl.ds(0, TAIL_ROWS)],
        )

        def zpad(r, carry):
            for k in range(DIM // 16):
                xbuf[0, r, pl.ds(k * 16, 16)] = zv
            return carry

        lax.fori_loop(TAIL_ROWS, CHUNK, zpad, 0)
        pltpu.sync_copy(xbuf.at[0], acc.at[idx_v.at[count - 1, 0]], add=True)

    plsc.subcore_barrier()

    pltpu.sync_copy(
        acc.at[pl.ds(s * 32, 32)], out_hbm.at[c, pl.ds(s * 32, 32)]
    )


@jax.jit
def _segment_sum_sc(x, batch_i32):
    mesh = plsc.VectorSubcoreMesh(
        core_axis_name="c", subcore_axis_name="s", num_cores=NC, num_subcores=NS
    )
    return pl.kernel(
        _sc_body,
        out_type=jax.ShapeDtypeStruct((NC, N_GRAPHS, DIM), jnp.float32),
        mesh=mesh,
        scratch_types=[
            pltpu.VMEM((IDX_PER_W,), jnp.int32),
            pltpu.VMEM((MAX_CH, 1, CHUNK), jnp.int32),
            pltpu.VMEM((NBUF, CHUNK, DIM), jnp.float32),
            pltpu.VMEM((32, DIM), jnp.float32),
            pltpu.VMEM_SHARED((N_GRAPHS, DIM), jnp.float32),
            pltpu.SemaphoreType.DMA((NBUF,)),
            pltpu.SemaphoreType.DMA((NBUF,)),
        ],
    )(x, batch_i32)


def _mlp_body(p_ref, w0_ref, b0_ref, w1_ref, b1_ref, w2_ref, b2_ref, o_ref):
    p = p_ref[0] + p_ref[1]
    h0 = jnp.maximum(
        jnp.dot(p, w0_ref[...], preferred_element_type=jnp.float32) + b0_ref[...],
        0.0,
    )
    h1 = jnp.maximum(
        jnp.dot(h0, w1_ref[...], preferred_element_type=jnp.float32) + b1_ref[...],
        0.0,
    )
    o_ref[...] = (
        jnp.sum(h1 * w2_ref[...], axis=1, keepdims=True) + b2_ref[...]
    )


@jax.jit
def _mlp_tc(partials, W0, b0, W1, b1, W2, b2):
    return pl.pallas_call(
        _mlp_body,
        out_shape=jax.ShapeDtypeStruct((N_GRAPHS, 1), jnp.float32),
    )(
        partials,
        W0.T,
        b0.reshape(1, -1),
        W1.T,
        b1.reshape(1, -1),
        W2,
        b2.reshape(1, 1),
    )


def kernel(x, batch, y, W0, b0, W1, b1, W2, b2):
    partials = _segment_sum_sc(x, batch.astype(jnp.int32))
    pred = _mlp_tc(partials, W0, b0, W1, b1, W2, b2)
    return (pred, y)


# speedup vs baseline: 1.0024x; 1.0024x over previous
import jax
import jax.numpy as jnp
from jax import lax
from jax.experimental import pallas as pl
from jax.experimental.pallas import tpu as pltpu
from jax.experimental.pallas import tpu_sc as plsc

N_ROWS = 100000
DIM = 128
N_GRAPHS = 512

NC = 2
NS = 16
NW = NC * NS

CHUNK = 128
NCH = (N_ROWS + CHUNK - 1) // CHUNK
TAIL_CHUNK = NCH - 1
TAIL_ROWS = N_ROWS - TAIL_CHUNK * CHUNK
MAX_CH = 25
IDX_PER_W = MAX_CH * CHUNK
TAIL_IDX = N_ROWS - (NW - 1) * IDX_PER_W
NBUF = 6


def _sc_body(x_hbm, b_hbm, out_hbm, idx_raw, idx_v, xbuf, zbuf, acc, gsems, ssems):
    c = lax.axis_index("c")
    s = lax.axis_index("s")
    w = s * NC + c

    start = w * MAX_CH
    count = jnp.clip(NCH - start, 0, MAX_CH)
    n_full = count - (w == NW - 1).astype(jnp.int32)

    def gather_start(j, b):
        pltpu.async_copy(
            x_hbm.at[pl.ds((start + j) * CHUNK, CHUNK)], xbuf.at[b], gsems.at[b]
        )

    def gather_wait(j, b):
        pltpu.make_async_copy(
            x_hbm.at[pl.ds((start + j) * CHUNK, CHUNK)], xbuf.at[b], gsems.at[b]
        ).wait()

    def scatter_start(j, b):
        pltpu.async_copy(xbuf.at[b], acc.at[idx_v.at[j, 0]], ssems.at[b], add=True)

    def scatter_wait(j, b):
        pltpu.make_async_copy(
            xbuf.at[b], acc.at[idx_v.at[j, 0]], ssems.at[b]
        ).wait()

    gather_start(0, 0)
    gather_start(1, 1)
    gather_start(2, 2)

    @pl.when(w == NW - 1)
    def _():
        pltpu.sync_copy(
            b_hbm.at[pl.ds((NW - 1) * IDX_PER_W, TAIL_IDX)],
            idx_raw.at[pl.ds(0, TAIL_IDX)],
        )

    @pl.when(w < NW - 1)
    def _():
        pltpu.sync_copy(b_hbm.at[pl.ds(w * IDX_PER_W, IDX_PER_W)], idx_raw)

    ziv = jnp.zeros((16,), jnp.int32)

    @pl.when(w == NW - 1)
    def _():
        def zidx(t, carry):
            idx_raw[pl.ds(TAIL_IDX + t * 16, 16)] = ziv
            return carry

        lax.fori_loop(0, (IDX_PER_W - TAIL_IDX) // 16, zidx, 0)

    def fixup(j, carry):
        for m in range(CHUNK // 16):
            idx_v[j, 0, pl.ds(m * 16, 16)] = idx_raw[pl.ds(j * CHUNK + m * 16, 16)]
        return carry

    lax.fori_loop(0, MAX_CH, fixup, 0)

    zv = jnp.zeros((16,), jnp.float32)

    def zrow(r, carry):
        for k in range(DIM // 16):
            zbuf[r, pl.ds(k * 16, 16)] = zv
        return carry

    lax.fori_loop(0, 32, zrow, 0)
    pltpu.sync_copy(zbuf, acc.at[pl.ds(s * 32, 32)])
    plsc.subcore_barrier()

    def body(j, carry):
        jn = j + 3

        @pl.when(jn < n_full)
        def _():
            bn = jn % NBUF

            @pl.when(jn >= NBUF)
            def _():
                scatter_wait(jn - NBUF, bn)

            gather_start(jn, bn)

        b = j % NBUF
        gather_wait(j, b)
        scatter_start(j, b)
        return carry

    lax.fori_loop(0, n_full, body, 0)

    def drain(t, carry):
        j = n_full - NBUF + t

        @pl.when(j >= 0)
        def _():
            scatter_wait(j, j % NBUF)

        return carry

    lax.fori_loop(0, NBUF, drain, 0)

    @pl.when(w == NW - 1)
    def _():
        pltpu.sync_copy(
            x_hbm.at[pl.ds(TAIL_CHUNK * CHUNK, TAIL_ROWS)],
            xbuf.at[0, pl.ds(0, TAIL_ROWS)],
        )

        def zpad(r, carry):
            for k in range(DIM // 16):
                xbuf[0, r, pl.ds(k * 16, 16)] = zv
            return carry

        lax.fori_loop(TAIL_ROWS, CHUNK, zpad, 0)
        pltpu.sync_copy(xbuf.at[0], acc.at[idx_v.at[count - 1, 0]], add=True)

    plsc.subcore_barrier()

    pltpu.sync_copy(
        acc.at[pl.ds(s * 32, 32)], out_hbm.at[c, pl.ds(s * 32, 32)]
    )


@jax.jit
def _segment_sum_sc(x, batch_i32):
    mesh = plsc.VectorSubcoreMesh(
        core_axis_name="c", subcore_axis_name="s", num_cores=NC, num_subcores=NS
    )
    return pl.kernel(
        _sc_body,
        out_type=jax.ShapeDtypeStruct((NC, N_GRAPHS, DIM), jnp.float32),
        mesh=mesh,
        scratch_types=[
            pltpu.VMEM((IDX_PER_W,), jnp.int32),
            pltpu.VMEM((MAX_CH, 1, CHUNK), jnp.int32),
            pltpu.VMEM((NBUF, CHUNK, DIM), jnp.float32),
            pltpu.VMEM((32, DIM), jnp.float32),
            pltpu.VMEM_SHARED((N_GRAPHS, DIM), jnp.float32),
            pltpu.SemaphoreType.DMA((NBUF,)),
            pltpu.SemaphoreType.DMA((NBUF,)),
        ],
    )(x, batch_i32)


def _mlp_body(p_ref, w0_ref, b0_ref, w1_ref, b1_ref, w2_ref, b2_ref, o_ref):
    p = p_ref[0] + p_ref[1]
    h0 = jnp.maximum(
        jnp.dot(p, w0_ref[...], preferred_element_type=jnp.float32) + b0_ref[...],
        0.0,
    )
    h1 = jnp.maximum(
        jnp.dot(h0, w1_ref[...], preferred_element_type=jnp.float32) + b1_ref[...],
        0.0,
    )
    o_ref[...] = (
        jnp.sum(h1 * w2_ref[...], axis=1, keepdims=True) + b2_ref[...]
    )


@jax.jit
def _mlp_tc(partials, W0, b0, W1, b1, W2, b2):
    return pl.pallas_call(
        _mlp_body,
        out_shape=jax.ShapeDtypeStruct((N_GRAPHS, 1), jnp.float32),
    )(
        partials,
        W0.T,
        b0.reshape(1, -1),
        W1.T,
        b1.reshape(1, -1),
        W2,
        b2.reshape(1, 1),
    )


def kernel(x, batch, y, W0, b0, W1, b1, W2, b2):
    partials = _segment_sum_sc(x, batch.astype(jnp.int32))
    pred = _mlp_tc(partials, W0, b0, W1, b1, W2, b2)
    return (pred, y)
